# baseline (speedup 1.0000x reference)
import functools

import jax
import jax.numpy as jnp
from jax import lax
from jax.experimental import pallas as pl
from jax.experimental.pallas import tpu as pltpu
from jax.experimental.pallas import tpu_sc as plsc

MAX_ROWS = 8192
D = 1024

NC = 2
NS = 16
NW = NC * NS
B_PER_W = MAX_ROWS // NW
CHUNK_S = 16
N_S = 8
CHUNK_T = 32
N_T = 4
S_ROWS = N_S * CHUNK_S
assert S_ROWS + N_T * CHUNK_T == B_PER_W

_mesh = plsc.VectorSubcoreMesh(core_axis_name="c", subcore_axis_name="s")


@functools.partial(
    pl.kernel,
    mesh=_mesh,
    out_type=jax.ShapeDtypeStruct((MAX_ROWS, D), jnp.float32),
    scratch_types=[
        pltpu.VMEM_SHARED((NS, 2, CHUNK_S, D), jnp.float32),
        pltpu.VMEM((2, CHUNK_T, D), jnp.float32),
        pltpu.VMEM((16,), jnp.int32),
        pltpu.VMEM((16, D), jnp.float32),
        pltpu.SemaphoreType.DMA,
        pltpu.SemaphoreType.DMA,
        pltpu.SemaphoreType.DMA,
        pltpu.SemaphoreType.DMA,
    ],
)
def _embed_rows(table_hbm, sl_hbm, out_hbm, sh, tb, sl_v, blk_v,
                sr_sem, sw_sem, tr_sem, tw_sem):
    sid = lax.axis_index("s")
    wid = sid * NC + lax.axis_index("c")
    base = wid * B_PER_W

    def _s_read(k):
        return pltpu.async_copy(
            table_hbm.at[pl.ds(base + k * CHUNK_S, CHUNK_S)],
            sh.at[sid, k % 2], sr_sem,
        )

    def _t_read(k):
        return pltpu.async_copy(
            table_hbm.at[pl.ds(base + S_ROWS + k * CHUNK_T, CHUNK_T)],
            tb.at[k % 2], tr_sem,
        )

    gs = _s_read(0)
    gt = _t_read(0)
    for k in range(N_S):
        j, run_t = divmod(k, 2)
        gs.wait()
        ws = pltpu.async_copy(
            sh.at[sid, k % 2],
            out_hbm.at[pl.ds(base + k * CHUNK_S, CHUNK_S)], sw_sem,
        )
        if k + 1 < N_S:
            gs = _s_read(k + 1)
        if run_t == 0:
            gt.wait()
            wt = pltpu.async_copy(
                tb.at[j % 2],
                out_hbm.at[pl.ds(base + S_ROWS + j * CHUNK_T, CHUNK_T)],
                tw_sem,
            )
            if j + 1 < N_T:
                gt = _t_read(j + 1)
        ws.wait()
        if run_t == 0:
            wt.wait()

    pltpu.sync_copy(sl_hbm, sl_v)
    sl = sl_v[...][0]
    last = jnp.maximum(sl - 1, 0)
    hi = base + B_PER_W
    lo = jnp.maximum(base, sl)
    start0 = jnp.minimum(lo - (lo % 16), hi - 16)
    n_blocks = jnp.where(lo < hi, (hi - start0) // 16, 0)

    def _fix(b, carry):
        start = pl.multiple_of(start0 + b * 16, 16)
        idx = jnp.minimum(start + lax.iota(jnp.int32, 16), last)
        pltpu.async_copy(table_hbm.at[idx], blk_v, sr_sem).wait()
        pltpu.sync_copy(blk_v, out_hbm.at[pl.ds(start, 16)])
        return carry

    lax.fori_loop(0, n_blocks, _fix, 0)


def kernel(seq_len, embedding_weight):
    sl_vec = jnp.full((16,), seq_len, dtype=jnp.int32)
    out = _embed_rows(embedding_weight, sl_vec)
    return out[None, :, :]


# speedup vs baseline: 1.0292x; 1.0292x over previous
import functools

import jax
import jax.numpy as jnp
from jax import lax
from jax.experimental import pallas as pl
from jax.experimental.pallas import tpu as pltpu
from jax.experimental.pallas import tpu_sc as plsc

MAX_ROWS = 8192
D = 1024

NC = 2
NS = 16
NW = NC * NS
B_PER_W = MAX_ROWS // NW
CHUNK_S = 32
N_S = 4
CHUNK_T = 32
N_T = 4
S_ROWS = N_S * CHUNK_S
assert S_ROWS + N_T * CHUNK_T == B_PER_W

_mesh = plsc.VectorSubcoreMesh(core_axis_name="c", subcore_axis_name="s")


@functools.partial(
    pl.kernel,
    mesh=_mesh,
    out_type=jax.ShapeDtypeStruct((MAX_ROWS, D), jnp.float32),
    scratch_types=[
        pltpu.VMEM_SHARED((NS, 2, CHUNK_S, D), jnp.float32),
        pltpu.VMEM((2, CHUNK_T, D), jnp.float32),
        pltpu.SemaphoreType.DMA,
        pltpu.SemaphoreType.DMA,
        pltpu.SemaphoreType.DMA,
        pltpu.SemaphoreType.DMA,
    ],
)
def _embed_rows(table_hbm, sl_hbm, out_hbm, sh, tb,
                sr_sem, sw_sem, tr_sem, tw_sem):
    sid = lax.axis_index("s")
    wid = sid * NC + lax.axis_index("c")
    base = wid * B_PER_W

    def _s_read(k):
        return pltpu.async_copy(
            table_hbm.at[pl.ds(base + k * CHUNK_S, CHUNK_S)],
            sh.at[sid, k % 2], sr_sem,
        )

    def _t_read(k):
        return pltpu.async_copy(
            table_hbm.at[pl.ds(base + S_ROWS + k * CHUNK_T, CHUNK_T)],
            tb.at[k % 2], tr_sem,
        )

    gs = _s_read(0)
    gt = _t_read(0)
    for k in range(N_S):
        gs.wait()
        ws = pltpu.async_copy(
            sh.at[sid, k % 2],
            out_hbm.at[pl.ds(base + k * CHUNK_S, CHUNK_S)], sw_sem,
        )
        if k + 1 < N_S:
            gs = _s_read(k + 1)
        gt.wait()
        wt = pltpu.async_copy(
            tb.at[k % 2],
            out_hbm.at[pl.ds(base + S_ROWS + k * CHUNK_T, CHUNK_T)],
            tw_sem,
        )
        if k + 1 < N_T:
            gt = _t_read(k + 1)
        ws.wait()
        wt.wait()

    pltpu.sync_copy(sl_hbm, tb.at[0, 0, pl.ds(0, 16)])
    sl = jnp.int32(tb[0, 0, pl.ds(0, 16)][0])
    last = jnp.maximum(sl - 1, 0)
    hi = base + B_PER_W
    lo = jnp.maximum(base, sl)
    start0 = jnp.minimum(lo - (lo % 16), hi - 16)
    n_blocks = jnp.where(lo < hi, (hi - start0) // 16, 0)

    def _fix(b, carry):
        start = pl.multiple_of(start0 + b * 16, 16)
        idx = jnp.minimum(start + lax.iota(jnp.int32, 16), last)
        stage = tb.at[0, pl.ds(0, 16)]
        pltpu.async_copy(table_hbm.at[idx], stage, sr_sem).wait()
        pltpu.sync_copy(stage, out_hbm.at[pl.ds(start, 16)])
        return carry

    lax.fori_loop(0, n_blocks, _fix, 0)


def kernel(seq_len, embedding_weight):
    sl_vec = jnp.full((16,), seq_len, dtype=jnp.float32)
    out = _embed_rows(embedding_weight, sl_vec)
    return out[None, :, :]
